# baseline (speedup 1.0000x reference)
import functools

import jax
import jax.numpy as jnp
from jax import lax
from jax.experimental import pallas as pl
from jax.experimental.pallas import tpu as pltpu
from jax.experimental.pallas import tpu_sc as plsc

L_TOTAL = 2_000_000
CHUNK = 3200
NCHUNK = L_TOTAL // CHUNK
NWORKER = 32
STEPS = -(-NCHUNK // NWORKER)
LANES = 16
UNROLL = 8


def kernel(seq):
    seq = seq.astype(jnp.int32)

    mesh = plsc.VectorSubcoreMesh(core_axis_name="c", subcore_axis_name="s")

    @functools.partial(
        pl.kernel,
        mesh=mesh,
        out_type=jax.ShapeDtypeStruct((4, L_TOTAL), jnp.float32),
        scratch_types=[
            pltpu.VMEM((2, CHUNK), jnp.int32),
            pltpu.VMEM((2, 4, CHUNK), jnp.float32),
            pltpu.SemaphoreType.DMA,
            pltpu.SemaphoreType.DMA,
            pltpu.SemaphoreType.DMA,
            pltpu.SemaphoreType.DMA,
        ],
    )
    def run(seq_hbm, out_hbm, seq_v, rows_v, in0, in1, out0, out1):
        wid = lax.axis_index("s") * 2 + lax.axis_index("c")
        insem = (in0, in1)
        outsem = (out0, out1)

        def in_copy(t, slot):
            base = (wid + t * NWORKER) * CHUNK
            return pltpu.make_async_copy(
                seq_hbm.at[pl.ds(base, CHUNK)], seq_v.at[slot], insem[slot])

        def out_copy(t, slot):
            base = (wid + t * NWORKER) * CHUNK
            return pltpu.make_async_copy(
                rows_v.at[slot], out_hbm.at[:, pl.ds(base, CHUNK)],
                outsem[slot])

        def compute(slot):
            @plsc.parallel_loop(0, CHUNK, step=LANES, unroll=UNROLL)
            def _(off):
                s = seq_v[slot, pl.ds(off, LANES)]
                fill = jnp.where(s == 4, jnp.float32(0.25),
                                 jnp.float32(0.0))
                for c in range(4):
                    rows_v[slot, c, pl.ds(off, LANES)] = jnp.where(
                        s == c, jnp.float32(1.0), fill)

        def valid(t):
            return wid + t * NWORKER < NCHUNK

        in_copy(0, 0).start()
        in_copy(1, 1).start()

        def round_(r, carry):
            for p in range(2):
                t = r * 2 + p
                pl.when(valid(t))(lambda: in_copy(t, p).wait())
                pl.when(r >= 1)(lambda: out_copy(t - 2, p).wait())
                compute(p)
                pl.when(valid(t))(lambda: out_copy(t, p).start())
                pl.when(jnp.logical_and(r < STEPS // 2 - 1, valid(t + 2)))(
                    lambda: in_copy(t + 2, p).start())
            return carry

        lax.fori_loop(0, STEPS // 2, round_, 0)

        out_copy(STEPS - 2, 0).wait()
        pl.when(valid(STEPS - 1))(lambda: out_copy(STEPS - 1, 1).wait())

    return run(seq)


# speedup vs baseline: 1.1098x; 1.1098x over previous
import functools

import jax
import jax.numpy as jnp
from jax import lax
from jax.experimental import pallas as pl
from jax.experimental.pallas import tpu as pltpu
from jax.experimental.pallas import tpu_sc as plsc

L_TOTAL = 2_000_000
CHUNK = 3200
NCHUNK = L_TOTAL // CHUNK
NWORKER = 32
STEPS = -(-NCHUNK // NWORKER)
LANES = 16
UNROLL = 4


def kernel(seq):
    seq = seq.astype(jnp.int32)

    mesh = plsc.VectorSubcoreMesh(core_axis_name="c", subcore_axis_name="s")

    @functools.partial(
        pl.kernel,
        mesh=mesh,
        out_type=jax.ShapeDtypeStruct((4, L_TOTAL), jnp.float32),
        scratch_types=[
            pltpu.VMEM((2, CHUNK), jnp.int32),
            pltpu.VMEM((2, 4, CHUNK), jnp.float32),
            pltpu.SemaphoreType.DMA,
            pltpu.SemaphoreType.DMA,
            pltpu.SemaphoreType.DMA,
            pltpu.SemaphoreType.DMA,
        ],
    )
    def run(seq_hbm, out_hbm, seq_v, rows_v, in0, in1, out0, out1):
        wid = lax.axis_index("s") * 2 + lax.axis_index("c")
        insem = (in0, in1)
        outsem = (out0, out1)

        def in_copy(t, slot):
            base = (wid + t * NWORKER) * CHUNK
            return pltpu.make_async_copy(
                seq_hbm.at[pl.ds(base, CHUNK)], seq_v.at[slot], insem[slot])

        def out_copy(t, slot):
            base = (wid + t * NWORKER) * CHUNK
            return pltpu.make_async_copy(
                rows_v.at[slot], out_hbm.at[:, pl.ds(base, CHUNK)],
                outsem[slot])

        def compute(slot):
            @plsc.parallel_loop(0, CHUNK, step=LANES, unroll=UNROLL)
            def _(off):
                s = seq_v[slot, pl.ds(off, LANES)]
                fill = jnp.where(s == 4, jnp.float32(0.25),
                                 jnp.float32(0.0))
                for c in range(4):
                    rows_v[slot, c, pl.ds(off, LANES)] = jnp.where(
                        s == c, jnp.float32(1.0), fill)

        def valid(t):
            return wid + t * NWORKER < NCHUNK

        in_copy(0, 0).start()
        in_copy(1, 1).start()

        def round_(r, carry):
            for p in range(2):
                t = r * 2 + p
                pl.when(valid(t))(lambda: in_copy(t, p).wait())
                pl.when(r >= 1)(lambda: out_copy(t - 2, p).wait())
                compute(p)
                pl.when(valid(t))(lambda: out_copy(t, p).start())
                pl.when(jnp.logical_and(r < STEPS // 2 - 1, valid(t + 2)))(
                    lambda: in_copy(t + 2, p).start())
            return carry

        lax.fori_loop(0, STEPS // 2, round_, 0)

        out_copy(STEPS - 2, 0).wait()
        pl.when(valid(STEPS - 1))(lambda: out_copy(STEPS - 1, 1).wait())

    return run(seq)
